# baseline (speedup 1.0000x reference)
import jax
import jax.numpy as jnp
from jax.experimental import pallas as pl
from jax.experimental.pallas import tpu as pltpu

_B = 16
_C = 256
_F = 250
_H = 4
_O = 250
_K = 8
_NEG = float("-inf")


def _eeg_gat_kernel(xf_hbm, wh_ref, att_s_ref, att_d_ref, bias_ref, wph_ref,
                    bp_ref, out_hbm, xbuf, obuf, in_sem, out_sem):
    f32 = jnp.float32

    def in_slot(b):
        return 2 if b == 0 else b % 2

    def in_copy(b):
        s = in_slot(b)
        return pltpu.make_async_copy(
            xf_hbm.at[pl.ds(b * _C, _C), :], xbuf.at[s], in_sem.at[s])

    def out_copy(b):
        s = b % 2
        return pltpu.make_async_copy(
            obuf.at[s], out_hbm.at[pl.ds(b * _C, _C), :], out_sem.at[s])

    in_copy(0).start()

    wc = jnp.zeros((_O, _F), f32)
    bvec = jnp.zeros((1, _O), f32)
    for hd in range(_H):
        wc = wc + jax.lax.dot_general(
            wph_ref[hd], wh_ref[hd], (((1,), (0,)), ((), ())),
            preferred_element_type=f32)
        bvec = bvec + jax.lax.dot_general(
            bias_ref[hd:hd + 1, :], wph_ref[hd], (((1,), (1,)), ((), ())),
            preferred_element_type=f32)
    add0 = bvec + bp_ref[...]

    acc = jnp.zeros((_C, _C), f32)
    for b in range(_B):
        if b + 1 < _B:
            in_copy(b + 1).start()
        in_copy(b).wait()
        xb = xbuf[in_slot(b)]
        mu = jnp.mean(xb, axis=1, keepdims=True)
        xc = xb - mu
        var = jnp.sum(xc * xc, axis=1, keepdims=True) * (1.0 / (_F - 1))
        xn = xc / (jnp.sqrt(var) + 1e-8)
        acc = acc + jax.lax.dot_general(
            xn, xn, (((1,), (1,)), ((), ())), preferred_element_type=f32)
        if b > 0:
            ob = jax.lax.dot_general(
                xb, wc, (((1,), (1,)), ((), ())),
                preferred_element_type=f32) + add0
            if b >= 3:
                out_copy(b - 2).wait()
            obuf[b % 2] = ob.astype(jnp.bfloat16)
            out_copy(b).start()
    out_copy(_B - 2).wait()
    out_copy(_B - 1).wait()

    adj = acc * (1.0 / (_B * _F))
    work = adj
    thr = jnp.max(work, axis=1, keepdims=True)
    for _ in range(_K - 1):
        work = jnp.where(work < thr, work, _NEG)
        thr = jnp.max(work, axis=1, keepdims=True)
    mask = jnp.logical_and(adj >= thr, adj != 0.0)

    rid = jax.lax.broadcasted_iota(jnp.int32, (_C, _C), 0)
    cid = jax.lax.broadcasted_iota(jnp.int32, (_C, _C), 1)
    eye = rid == cid

    x0 = xbuf[2]
    final0 = jnp.broadcast_to(bp_ref[...], (_C, _O)).astype(f32)
    for hd in range(_H):
        h0h = jax.lax.dot_general(
            x0, wh_ref[hd], (((1,), (1,)), ((), ())),
            preferred_element_type=f32)
        asc = jax.lax.dot_general(
            h0h, att_s_ref[hd:hd + 1, :], (((1,), (1,)), ((), ())),
            preferred_element_type=f32)
        adt = jax.lax.dot_general(
            att_d_ref[hd:hd + 1, :], h0h, (((1,), (1,)), ((), ())),
            preferred_element_type=f32)
        logit = asc + adt
        logit = jnp.where(logit > 0, logit, 0.2 * logit)
        lmask = jnp.where(mask, logit, _NEG)
        ldiag = jnp.max(jnp.where(eye, logit, _NEG), axis=0, keepdims=True)
        m = jnp.maximum(jnp.max(lmask, axis=0, keepdims=True), ldiag)
        e = jnp.exp(lmask - m)
        es = jnp.exp(ldiag - m)
        denom = jnp.sum(e, axis=0, keepdims=True) + es
        attw = (e + jnp.where(eye, es, 0.0)) / denom
        attn = jax.lax.dot_general(
            attw, h0h, (((0,), (0,)), ((), ())), preferred_element_type=f32)
        final0 = final0 + jax.lax.dot_general(
            attn + bias_ref[hd:hd + 1, :], wph_ref[hd], (((1,), (1,)), ((), ())),
            preferred_element_type=f32)
    obuf[0] = final0.astype(jnp.bfloat16)
    out_copy(0).start()
    out_copy(0).wait()


def kernel(x, W, att_src, att_dst, bias, Wp, bp):
    bf16 = jnp.bfloat16
    xf = x.reshape(_B * _C, _F)
    wh = W.reshape(_H, _O, _F).astype(bf16)
    wph = Wp.reshape(_O, _H, _O).transpose(1, 0, 2).astype(bf16)
    att_s = att_src.reshape(_H, _O)
    att_d = att_dst.reshape(_H, _O)
    bias_h = bias.reshape(_H, _O)
    bp2 = bp.reshape(1, _O)
    vspec = pl.BlockSpec(memory_space=pltpu.VMEM)
    out = pl.pallas_call(
        _eeg_gat_kernel,
        in_specs=[pl.BlockSpec(memory_space=pltpu.HBM),
                  vspec, vspec, vspec, vspec, vspec, vspec],
        out_specs=pl.BlockSpec(memory_space=pltpu.HBM),
        out_shape=jax.ShapeDtypeStruct((_B * _C, _O), bf16),
        scratch_shapes=[
            pltpu.VMEM((3, _C, _F), jnp.float32),
            pltpu.VMEM((2, _C, _O), bf16),
            pltpu.SemaphoreType.DMA((3,)),
            pltpu.SemaphoreType.DMA((2,)),
        ],
    )(xf, wh, att_s, att_d, bias_h, wph, bp2)
    return out.astype(jnp.float32).reshape(_B, 1, _C, _O)


# speedup vs baseline: 1.2756x; 1.2756x over previous
import jax
import jax.numpy as jnp
from jax.experimental import pallas as pl
from jax.experimental.pallas import tpu as pltpu

_B = 16
_C = 256
_F = 250
_H = 4
_O = 250
_K = 8
_NEG = float("-inf")


def _eeg_gat_kernel(xf_ref, wh_ref, att_s_ref, att_d_ref, bias_ref, wph_ref,
                    bp_ref, out_ref):
    f32 = jnp.float32
    xf = xf_ref[...]
    x0 = xf[0:_C, :]

    acc = jnp.zeros((_C, _C), f32)
    for b in range(_B):
        xb = xf[b * _C:(b + 1) * _C, :]
        mu = jnp.mean(xb, axis=1, keepdims=True)
        xc = xb - mu
        var = jnp.sum(xc * xc, axis=1, keepdims=True) * (1.0 / (_F - 1))
        xn = xc / (jnp.sqrt(var) + 1e-8)
        acc = acc + jax.lax.dot_general(
            xn, xn, (((1,), (1,)), ((), ())), preferred_element_type=f32)
    adj = acc * (1.0 / (_B * _F))

    work = adj
    thr = jnp.max(work, axis=1, keepdims=True)
    for _ in range(_K - 1):
        work = jnp.where(work < thr, work, _NEG)
        thr = jnp.max(work, axis=1, keepdims=True)
    mask = jnp.logical_and(adj >= thr, adj != 0.0)

    rid = jax.lax.broadcasted_iota(jnp.int32, (_C, _C), 0)
    cid = jax.lax.broadcasted_iota(jnp.int32, (_C, _C), 1)
    eye = rid == cid

    final0 = jnp.broadcast_to(bp_ref[...], (_C, _O)).astype(f32)
    wc = jnp.zeros((_O, _F), f32)
    bvec = jnp.zeros((1, _O), f32)
    for hd in range(_H):
        wh = wh_ref[hd]
        wph = wph_ref[hd]
        h0h = jax.lax.dot_general(
            x0, wh, (((1,), (1,)), ((), ())), preferred_element_type=f32)
        asc = jax.lax.dot_general(
            h0h, att_s_ref[hd:hd + 1, :], (((1,), (1,)), ((), ())),
            preferred_element_type=f32)
        adt = jax.lax.dot_general(
            att_d_ref[hd:hd + 1, :], h0h, (((1,), (1,)), ((), ())),
            preferred_element_type=f32)
        logit = asc + adt
        logit = jnp.where(logit > 0, logit, 0.2 * logit)
        lmask = jnp.where(mask, logit, _NEG)
        ldiag = jnp.max(jnp.where(eye, logit, _NEG), axis=0, keepdims=True)
        m = jnp.maximum(jnp.max(lmask, axis=0, keepdims=True), ldiag)
        e = jnp.exp(lmask - m)
        es = jnp.exp(ldiag - m)
        denom = jnp.sum(e, axis=0, keepdims=True) + es
        attw = (e + jnp.where(eye, es, 0.0)) / denom
        attn = jax.lax.dot_general(
            attw, h0h, (((0,), (0,)), ((), ())), preferred_element_type=f32)
        final0 = final0 + jax.lax.dot_general(
            attn + bias_ref[hd:hd + 1, :], wph, (((1,), (1,)), ((), ())),
            preferred_element_type=f32)
        wc = wc + jax.lax.dot_general(
            wph, wh, (((1,), (0,)), ((), ())), preferred_element_type=f32)
        bvec = bvec + jax.lax.dot_general(
            bias_ref[hd:hd + 1, :], wph, (((1,), (1,)), ((), ())),
            preferred_element_type=f32)

    out_all = jax.lax.dot_general(
        xf, wc, (((1,), (1,)), ((), ())),
        preferred_element_type=f32) + bvec + bp_ref[...]
    out_ref[...] = out_all.astype(jnp.bfloat16)
    out_ref[0:_C, :] = final0.astype(jnp.bfloat16)


def kernel(x, W, att_src, att_dst, bias, Wp, bp):
    bf16 = jnp.bfloat16
    xf = x.reshape(_B * _C, _F)
    wh = W.reshape(_H, _O, _F).astype(bf16)
    wph = Wp.reshape(_O, _H, _O).transpose(1, 0, 2).astype(bf16)
    att_s = att_src.reshape(_H, _O)
    att_d = att_dst.reshape(_H, _O)
    bias_h = bias.reshape(_H, _O)
    bp2 = bp.reshape(1, _O)
    out = pl.pallas_call(
        _eeg_gat_kernel,
        out_shape=jax.ShapeDtypeStruct((_B * _C, _O), bf16),
        compiler_params=pltpu.CompilerParams(
            allow_input_fusion=[True] * 7,
            skip_device_barrier=True,
        ),
    )(xf, wh, att_s, att_d, bias_h, wph, bp2)
    return out.astype(jnp.float32).reshape(_B, 1, _C, _O)


# speedup vs baseline: 1.3155x; 1.0313x over previous
import jax
import jax.numpy as jnp
from jax.experimental import pallas as pl
from jax.experimental.pallas import tpu as pltpu

_B = 16
_C = 256
_F = 250
_H = 4
_O = 250
_K = 8
_HB = _B // 2
_R = _HB * _C
_NEG = float("-inf")


def _eeg_gat_kernel(xf_hbm, wh_ref, att_s_ref, att_d_ref, bias_ref, wph_ref,
                    bp_ref, out_hbm, xbuf, obuf, in_sem, out_sem):
    f32 = jnp.float32

    def in_copy(half):
        return pltpu.make_async_copy(
            xf_hbm.at[pl.ds(half * _R, _R), :], xbuf.at[half],
            in_sem.at[half])

    def out_copy(half):
        return pltpu.make_async_copy(
            obuf.at[half], out_hbm.at[pl.ds(half * _R, _R), :],
            out_sem.at[half])

    in_copy(0).start()
    in_copy(1).start()

    wc = jnp.zeros((_O, _F), f32)
    bvec = jnp.zeros((1, _O), f32)
    for hd in range(_H):
        wc = wc + jax.lax.dot_general(
            wph_ref[hd], wh_ref[hd], (((1,), (0,)), ((), ())),
            preferred_element_type=f32)
        bvec = bvec + jax.lax.dot_general(
            bias_ref[hd:hd + 1, :], wph_ref[hd], (((1,), (1,)), ((), ())),
            preferred_element_type=f32)
    add0 = bvec + bp_ref[...]

    acc = jnp.zeros((_C, _C), f32)
    for half in range(2):
        in_copy(half).wait()
        xh = xbuf[half]
        for b in range(_HB):
            xb = xh[b * _C:(b + 1) * _C, :]
            mu = jnp.mean(xb, axis=1, keepdims=True)
            xc = xb - mu
            var = jnp.sum(xc * xc, axis=1, keepdims=True) * (1.0 / (_F - 1))
            xn = xc / (jnp.sqrt(var) + 1e-8)
            acc = acc + jax.lax.dot_general(
                xn, xn, (((1,), (1,)), ((), ())), preferred_element_type=f32)
        ob = jax.lax.dot_general(
            xh, wc, (((1,), (1,)), ((), ())),
            preferred_element_type=f32) + add0
        obuf[half] = ob.astype(jnp.bfloat16)
        out_copy(half).start()

    adj = acc * (1.0 / (_B * _F))
    work = adj
    thr = jnp.max(work, axis=1, keepdims=True)
    for _ in range(_K - 1):
        work = jnp.where(work < thr, work, _NEG)
        thr = jnp.max(work, axis=1, keepdims=True)
    mask = jnp.logical_and(adj >= thr, adj != 0.0)

    rid = jax.lax.broadcasted_iota(jnp.int32, (_C, _C), 0)
    cid = jax.lax.broadcasted_iota(jnp.int32, (_C, _C), 1)
    eye = rid == cid

    x0 = xbuf[0, 0:_C, :]
    final0 = jnp.broadcast_to(bp_ref[...], (_C, _O)).astype(f32)
    for hd in range(_H):
        h0h = jax.lax.dot_general(
            x0, wh_ref[hd], (((1,), (1,)), ((), ())),
            preferred_element_type=f32)
        asc = jax.lax.dot_general(
            h0h, att_s_ref[hd:hd + 1, :], (((1,), (1,)), ((), ())),
            preferred_element_type=f32)
        adt = jax.lax.dot_general(
            att_d_ref[hd:hd + 1, :], h0h, (((1,), (1,)), ((), ())),
            preferred_element_type=f32)
        logit = asc + adt
        logit = jnp.where(logit > 0, logit, 0.2 * logit)
        lmask = jnp.where(mask, logit, _NEG)
        ldiag = jnp.max(jnp.where(eye, logit, _NEG), axis=0, keepdims=True)
        m = jnp.maximum(jnp.max(lmask, axis=0, keepdims=True), ldiag)
        e = jnp.exp(lmask - m)
        es = jnp.exp(ldiag - m)
        denom = jnp.sum(e, axis=0, keepdims=True) + es
        attw = (e + jnp.where(eye, es, 0.0)) / denom
        attn = jax.lax.dot_general(
            attw, h0h, (((0,), (0,)), ((), ())), preferred_element_type=f32)
        final0 = final0 + jax.lax.dot_general(
            attn + bias_ref[hd:hd + 1, :], wph_ref[hd], (((1,), (1,)), ((), ())),
            preferred_element_type=f32)

    out_copy(0).wait()
    obuf[0, 0:_C, :] = final0.astype(jnp.bfloat16)
    blk0 = pltpu.make_async_copy(
        obuf.at[0, pl.ds(0, _C), :], out_hbm.at[pl.ds(0, _C), :],
        in_sem.at[0])
    blk0.start()
    blk0.wait()
    out_copy(1).wait()


def kernel(x, W, att_src, att_dst, bias, Wp, bp):
    bf16 = jnp.bfloat16
    xf = x.reshape(_B * _C, _F)
    wh = W.reshape(_H, _O, _F).astype(bf16)
    wph = Wp.reshape(_O, _H, _O).transpose(1, 0, 2).astype(bf16)
    att_s = att_src.reshape(_H, _O)
    att_d = att_dst.reshape(_H, _O)
    bias_h = bias.reshape(_H, _O)
    bp2 = bp.reshape(1, _O)
    vspec = pl.BlockSpec(memory_space=pltpu.VMEM)
    out = pl.pallas_call(
        _eeg_gat_kernel,
        in_specs=[pl.BlockSpec(memory_space=pltpu.HBM),
                  vspec, vspec, vspec, vspec, vspec, vspec],
        out_specs=pl.BlockSpec(memory_space=pltpu.HBM),
        out_shape=jax.ShapeDtypeStruct((_B * _C, _O), bf16),
        scratch_shapes=[
            pltpu.VMEM((2, _R, _F), jnp.float32),
            pltpu.VMEM((2, _R, _O), bf16),
            pltpu.SemaphoreType.DMA((2,)),
            pltpu.SemaphoreType.DMA((2,)),
        ],
        compiler_params=pltpu.CompilerParams(
            allow_input_fusion=[True] * 7,
            skip_device_barrier=True,
        ),
    )(xf, wh, att_s, att_d, bias_h, wph, bp2)
    return out.astype(jnp.float32).reshape(_B, 1, _C, _O)
